# baseline (speedup 1.0000x reference)
import functools

import jax
import jax.numpy as jnp
from jax import lax
from jax.experimental import pallas as pl
from jax.experimental.pallas import tpu as pltpu
from jax.experimental.pallas import tpu_sc as plsc

BATCH = 4096
SEQ = 2050
NUM_BANDS = 16
BAND_ROWS = 2048 // NUM_BANDS
HALF = BATCH // 2
CHUNK = 8
CHUNKS = BAND_ROWS // CHUNK
VECS = HALF // 16


def _make_sc_kernel():
    mesh = plsc.VectorSubcoreMesh(core_axis_name="c", subcore_axis_name="s")
    out = jax.ShapeDtypeStruct((SEQ, BATCH), jnp.int32)
    buf = pltpu.VMEM((CHUNK, HALF), jnp.int32)

    @functools.partial(
        pl.kernel,
        mesh=mesh,
        out_type=out,
        scratch_types=[buf] * 5 + [pltpu.SemaphoreType.DMA((3,)),
                                   pltpu.SemaphoreType.DMA((2,))],
        compiler_params=pltpu.CompilerParams(needs_layout_passes=False),
    )
    def run(x_hbm, r_hbm, in0, in1, in2, r0, r1, si, so):
        ins, rs = (in0, in1, in2), (r0, r1)
        wid = lax.axis_index("s") * 2 + lax.axis_index("c")
        band = wid >> 1
        col0 = (wid & 1) * HALF
        row_base = band * BAND_ROWS
        zero = jnp.zeros((16,), jnp.int32)

        def in_cp(t, b):
            return pltpu.make_async_copy(
                x_hbm.at[pl.ds(row_base + t * CHUNK, CHUNK),
                         pl.ds(col0, HALF)],
                ins[b], si.at[b])

        def outr_cp(t, b):
            return pltpu.make_async_copy(
                rs[b], r_hbm.at[pl.ds(row_base + t * CHUNK, CHUNK),
                                pl.ds(col0, HALF)], so.at[b])

        def compute(bi, bp, bo):
            in_v, prev_v = ins[bi], ins[bp]
            r_v = rs[bo]

            def vec_body(i, _):
                off = i * 16
                x1 = in_v[1, pl.ds(off, 16)]
                x5 = in_v[5, pl.ds(off, 16)]
                xp5 = prev_v[5, pl.ds(off, 16)]
                rbt_a = jnp.where(x1 >= 2, x1, zero)
                rbt_b = jnp.where(x5 >= 2, x5, zero)
                rbt_c = jnp.where(xp5 >= 2, xp5, zero)
                r_v[0, pl.ds(off, 16)] = rbt_c
                r_v[1, pl.ds(off, 16)] = zero
                r_v[2, pl.ds(off, 16)] = rbt_a
                r_v[3, pl.ds(off, 16)] = rbt_a
                r_v[4, pl.ds(off, 16)] = rbt_a
                r_v[5, pl.ds(off, 16)] = zero
                r_v[6, pl.ds(off, 16)] = rbt_b
                r_v[7, pl.ds(off, 16)] = rbt_b
                return 0

            lax.fori_loop(0, VECS, vec_body, 0)

        prev_base = jnp.where(band == 0, 0, row_base - CHUNK)
        pltpu.async_copy(
            x_hbm.at[pl.ds(prev_base, CHUNK), pl.ds(col0, HALF)],
            ins[2], si.at[2]).wait()
        in_cp(0, 0).start()

        @pl.loop(0, 18, step=6)
        def _(ts):
            for b in range(6):
                t = ts + b
                bi, bo = b % 3, b % 2

                @pl.when(t < CHUNKS)
                def _():
                    @pl.when(t + 1 < CHUNKS)
                    def _():
                        in_cp(t + 1, (bi + 1) % 3).start()

                    in_cp(t, bi).wait()

                    @pl.when(t >= 2)
                    def _():
                        outr_cp(t - 2, bo).wait()

                    compute(bi, (bi + 2) % 3, bo)

                    @pl.when((band == 0) & (t == 0))
                    def _():
                        def z0(i, _):
                            rs[0][0, pl.ds(i * 16, 16)] = zero
                            return 0

                        lax.fori_loop(0, VECS, z0, 0)

                    outr_cp(t, bo).start()

        outr_cp(CHUNKS - 2, 0).wait()
        outr_cp(CHUNKS - 1, 1).wait()

        @pl.when(band == NUM_BANDS - 1)
        def _():
            def tail_body(i, _):
                off = i * 16
                tok = ins[0][5, pl.ds(off, 16)]
                rs[0][0, pl.ds(off, 16)] = jnp.where(tok >= 2, tok, zero)
                rs[0][1, pl.ds(off, 16)] = zero
                return 0

            lax.fori_loop(0, VECS, tail_body, 0)
            pltpu.sync_copy(rs[0].at[pl.ds(0, 2)],
                            r_hbm.at[pl.ds(2048, 2), pl.ds(col0, HALF)])

    return run


def _pos_tc_body(x_ref, p_ref):
    cols = x_ref.shape[1]
    j = lax.broadcasted_iota(jnp.int32, (SEQ, cols), 0)
    pre = ((j + 3) & 3) + 1
    inb = (j >= 1) & (j <= SEQ - 2)
    x = x_ref[...]
    p_ref[...] = jnp.where((x >= 2) & inb, pre, 0)


def _make_tc_kernel():
    cols = 512
    grid = BATCH // cols
    return pl.pallas_call(
        _pos_tc_body,
        grid=(grid,),
        in_specs=[pl.BlockSpec((SEQ, cols), lambda i: (0, i))],
        out_specs=pl.BlockSpec((SEQ, cols), lambda i: (0, i)),
        out_shape=jax.ShapeDtypeStruct((SEQ, BATCH), jnp.int32),
        compiler_params=pltpu.CompilerParams(
            dimension_semantics=("arbitrary",)),
    )


_RUN_SC = _make_sc_kernel()
_RUN_TC = _make_tc_kernel()


def kernel(input_id_sequence):
    xt = input_id_sequence.T
    rt = _RUN_SC(xt)
    pt = _RUN_TC(xt)
    return (pt.T, rt.T)


# speedup vs baseline: 1.0168x; 1.0168x over previous
import functools

import jax
import jax.numpy as jnp
from jax import lax
from jax.experimental import pallas as pl
from jax.experimental.pallas import tpu as pltpu
from jax.experimental.pallas import tpu_sc as plsc

BATCH = 4096
SEQ = 2050
NUM_BANDS = 16
BAND_ROWS = 2048 // NUM_BANDS
HALF = BATCH // 2
CHUNK = 8
CHUNKS = BAND_ROWS // CHUNK
VECS = HALF // 16
POS_CONST = (4, 1, 2, 3, 4, 1, 2, 3)


def _make_kernel():
    mesh = plsc.VectorSubcoreMesh(core_axis_name="c", subcore_axis_name="s")
    out = jax.ShapeDtypeStruct((SEQ, BATCH), jnp.int32)
    buf = pltpu.VMEM((CHUNK, HALF), jnp.int32)

    @functools.partial(
        pl.kernel,
        mesh=mesh,
        out_type=[out, out],
        scratch_types=[buf] * 7 + [pltpu.SemaphoreType.DMA((3, 2)),
                                   pltpu.SemaphoreType.DMA((2, 2))],
        compiler_params=pltpu.CompilerParams(needs_layout_passes=False),
    )
    def run(x_hbm, p_hbm, r_hbm, in0, in1, in2, p0, p1, r0, r1, si, so):
        ins, ps, rs = (in0, in1, in2), (p0, p1), (r0, r1)
        wid = lax.axis_index("s") * 2 + lax.axis_index("c")
        band = wid >> 1
        col0 = (wid & 1) * HALF
        row_base = band * BAND_ROWS
        zero = jnp.zeros((16,), jnp.int32)

        Q = HALF // 2

        def in_cps(t, b):
            rows = pl.ds(row_base + t * CHUNK, CHUNK)
            return [pltpu.make_async_copy(
                        x_hbm.at[rows, pl.ds(col0 + h * Q, Q)],
                        ins[b].at[:, pl.ds(h * Q, Q)], si.at[b, h])
                    for h in range(2)]

        def outp_cps(t, b):
            rows = pl.ds(row_base + t * CHUNK, CHUNK)
            return [pltpu.make_async_copy(
                        ps[b].at[:, pl.ds(h * Q, Q)],
                        p_hbm.at[rows, pl.ds(col0 + h * Q, Q)], so.at[b, h])
                    for h in range(2)]

        def outr_cps(t, b):
            rows = pl.ds(row_base + t * CHUNK, CHUNK)
            return [pltpu.make_async_copy(
                        rs[b].at[:, pl.ds(h * Q, Q)],
                        r_hbm.at[rows, pl.ds(col0 + h * Q, Q)], so.at[b, h])
                    for h in range(2)]

        def compute(bi, bp, bo):
            in_v, prev_v = ins[bi], ins[bp]
            p_v, r_v = ps[bo], rs[bo]

            def vec_body(i, _):
                off = i * 16
                xs = [in_v[rr, pl.ds(off, 16)] for rr in range(CHUNK)]
                xp5 = prev_v[5, pl.ds(off, 16)]
                rbt_a = jnp.where(xs[1] >= 2, xs[1], zero)
                rbt_b = jnp.where(xs[5] >= 2, xs[5], zero)
                rbt_c = jnp.where(xp5 >= 2, xp5, zero)
                for rr in range(CHUNK):
                    p_v[rr, pl.ds(off, 16)] = jnp.where(
                        xs[rr] >= 2, POS_CONST[rr], 0)
                r_v[0, pl.ds(off, 16)] = rbt_c
                r_v[1, pl.ds(off, 16)] = zero
                r_v[2, pl.ds(off, 16)] = rbt_a
                r_v[3, pl.ds(off, 16)] = rbt_a
                r_v[4, pl.ds(off, 16)] = rbt_a
                r_v[5, pl.ds(off, 16)] = zero
                r_v[6, pl.ds(off, 16)] = rbt_b
                r_v[7, pl.ds(off, 16)] = rbt_b
                return 0

            lax.fori_loop(0, VECS, vec_body, 0)

        prev_base = jnp.where(band == 0, 0, row_base - CHUNK)
        for h in range(2):
            pltpu.async_copy(
                x_hbm.at[pl.ds(prev_base, CHUNK),
                         pl.ds(col0 + h * (HALF // 2), HALF // 2)],
                ins[2].at[:, pl.ds(h * (HALF // 2), HALF // 2)],
                si.at[2, h])
        for h in range(2):
            pltpu.make_async_copy(
                x_hbm.at[pl.ds(prev_base, CHUNK),
                         pl.ds(col0 + h * (HALF // 2), HALF // 2)],
                ins[2].at[:, pl.ds(h * (HALF // 2), HALF // 2)],
                si.at[2, h]).wait()
        for c in in_cps(0, 0):
            c.start()

        @pl.loop(0, 18, step=6)
        def _(ts):
            for b in range(6):
                t = ts + b
                bi, bo = b % 3, b % 2

                @pl.when(t < CHUNKS)
                def _():
                    @pl.when(t + 1 < CHUNKS)
                    def _():
                        for c in in_cps(t + 1, (bi + 1) % 3):
                            c.start()

                    for c in in_cps(t, bi):
                        c.wait()

                    @pl.when(t >= 2)
                    def _():
                        for c in outp_cps(t - 2, bo):
                            c.wait()
                        for c in outr_cps(t - 2, bo):
                            c.wait()

                    compute(bi, (bi + 2) % 3, bo)

                    @pl.when((band == 0) & (t == 0))
                    def _():
                        def z0(i, _):
                            ps[0][0, pl.ds(i * 16, 16)] = zero
                            rs[0][0, pl.ds(i * 16, 16)] = zero
                            return 0

                        lax.fori_loop(0, VECS, z0, 0)

                    for c in outp_cps(t, bo):
                        c.start()
                    for c in outr_cps(t, bo):
                        c.start()

        for c in outp_cps(CHUNKS - 2, 0) + outr_cps(CHUNKS - 2, 0):
            c.wait()
        for c in outp_cps(CHUNKS - 1, 1) + outr_cps(CHUNKS - 1, 1):
            c.wait()

        @pl.when(band == NUM_BANDS - 1)
        def _():
            pltpu.async_copy(
                x_hbm.at[pl.ds(2048, 2), pl.ds(col0, HALF)],
                ins[1].at[pl.ds(0, 2)], si.at[1, 0]).wait()

            def tail_body(i, _):
                off = i * 16
                x = ins[1][0, pl.ds(off, 16)]
                tok = ins[0][5, pl.ds(off, 16)]
                ps[0][0, pl.ds(off, 16)] = jnp.where(x >= 2, 4, 0)
                rs[0][0, pl.ds(off, 16)] = jnp.where(tok >= 2, tok, zero)
                ps[0][1, pl.ds(off, 16)] = zero
                rs[0][1, pl.ds(off, 16)] = zero
                return 0

            lax.fori_loop(0, VECS, tail_body, 0)
            pltpu.sync_copy(ps[0].at[pl.ds(0, 2)],
                            p_hbm.at[pl.ds(2048, 2), pl.ds(col0, HALF)])
            pltpu.sync_copy(rs[0].at[pl.ds(0, 2)],
                            r_hbm.at[pl.ds(2048, 2), pl.ds(col0, HALF)])

    return run


_RUN = _make_kernel()


def kernel(input_id_sequence):
    pt, rt = _RUN(input_id_sequence.T)
    return (pt.T, rt.T)


# speedup vs baseline: 1.0217x; 1.0048x over previous
import functools

import jax
import jax.numpy as jnp
from jax import lax
from jax.experimental import pallas as pl
from jax.experimental.pallas import tpu as pltpu
from jax.experimental.pallas import tpu_sc as plsc

BATCH = 4096
SEQ = 2050
NUM_BANDS = 16
BAND_ROWS = 2048 // NUM_BANDS
HALF = BATCH // 2
CHUNK = 8
CHUNKS = BAND_ROWS // CHUNK
VECS = HALF // 16
POS_CONST = (4, 1, 2, 3, 4, 1, 2, 3)


def _make_kernel():
    mesh = plsc.VectorSubcoreMesh(core_axis_name="c", subcore_axis_name="s")
    out = jax.ShapeDtypeStruct((SEQ, BATCH), jnp.int32)
    buf = pltpu.VMEM((CHUNK, HALF), jnp.int32)

    @functools.partial(
        pl.kernel,
        mesh=mesh,
        out_type=[out, out],
        scratch_types=[buf] * 7 + [pltpu.SemaphoreType.DMA((3,)),
                                   pltpu.SemaphoreType.DMA((2,))],
        compiler_params=pltpu.CompilerParams(needs_layout_passes=False),
    )
    def run(x_hbm, p_hbm, r_hbm, in0, in1, in2, p0, p1, r0, r1, si, so):
        ins, ps, rs = (in0, in1, in2), (p0, p1), (r0, r1)
        wid = lax.axis_index("s") * 2 + lax.axis_index("c")
        band = wid >> 1
        col0 = (wid & 1) * HALF
        row_base = band * BAND_ROWS
        zero = jnp.zeros((16,), jnp.int32)

        def in_cp(t, b):
            return pltpu.make_async_copy(
                x_hbm.at[pl.ds(row_base + t * CHUNK, CHUNK),
                         pl.ds(col0, HALF)],
                ins[b], si.at[b])

        def outp_cp(t, b):
            return pltpu.make_async_copy(
                ps[b], p_hbm.at[pl.ds(row_base + t * CHUNK, CHUNK),
                                pl.ds(col0, HALF)], so.at[b])

        def outr_cp(t, b):
            return pltpu.make_async_copy(
                rs[b], r_hbm.at[pl.ds(row_base + t * CHUNK, CHUNK),
                                pl.ds(col0, HALF)], so.at[b])

        def compute(bi, bp, bo):
            in_v, prev_v = ins[bi], ins[bp]
            p_v, r_v = ps[bo], rs[bo]

            @plsc.parallel_loop(0, VECS, unroll=4)
            def vec_body(i):
                off = i * 16
                xs = [in_v[rr, pl.ds(off, 16)] for rr in range(CHUNK)]
                xp5 = prev_v[5, pl.ds(off, 16)]
                rbt_a = jnp.where(xs[1] >= 2, xs[1], zero)
                rbt_b = jnp.where(xs[5] >= 2, xs[5], zero)
                rbt_c = jnp.where(xp5 >= 2, xp5, zero)
                for rr in range(CHUNK):
                    p_v[rr, pl.ds(off, 16)] = jnp.where(
                        xs[rr] >= 2, POS_CONST[rr], 0)
                r_v[0, pl.ds(off, 16)] = rbt_c
                r_v[1, pl.ds(off, 16)] = zero
                r_v[2, pl.ds(off, 16)] = rbt_a
                r_v[3, pl.ds(off, 16)] = rbt_a
                r_v[4, pl.ds(off, 16)] = rbt_a
                r_v[5, pl.ds(off, 16)] = zero
                r_v[6, pl.ds(off, 16)] = rbt_b
                r_v[7, pl.ds(off, 16)] = rbt_b

        prev_base = jnp.where(band == 0, 0, row_base - CHUNK)
        pltpu.async_copy(
            x_hbm.at[pl.ds(prev_base, CHUNK), pl.ds(col0, HALF)],
            ins[2], si.at[2]).wait()
        in_cp(0, 0).start()

        @pl.loop(0, 18, step=6)
        def _(ts):
            for b in range(6):
                t = ts + b
                bi, bo = b % 3, b % 2

                @pl.when(t < CHUNKS)
                def _():
                    @pl.when(t + 1 < CHUNKS)
                    def _():
                        in_cp(t + 1, (bi + 1) % 3).start()

                    in_cp(t, bi).wait()

                    @pl.when(t >= 2)
                    def _():
                        outp_cp(t - 2, bo).wait()
                        outr_cp(t - 2, bo).wait()

                    compute(bi, (bi + 2) % 3, bo)

                    @pl.when((band == 0) & (t == 0))
                    def _():
                        @plsc.parallel_loop(0, VECS, unroll=4)
                        def z0(i):
                            ps[0][0, pl.ds(i * 16, 16)] = zero
                            rs[0][0, pl.ds(i * 16, 16)] = zero

                    outp_cp(t, bo).start()
                    outr_cp(t, bo).start()

        outp_cp(CHUNKS - 2, 0).wait()
        outr_cp(CHUNKS - 2, 0).wait()
        outp_cp(CHUNKS - 1, 1).wait()
        outr_cp(CHUNKS - 1, 1).wait()

        @pl.when(band == NUM_BANDS - 1)
        def _():
            pltpu.async_copy(
                x_hbm.at[pl.ds(2048, 2), pl.ds(col0, HALF)],
                ins[1].at[pl.ds(0, 2)], si.at[1]).wait()

            @plsc.parallel_loop(0, VECS, unroll=4)
            def tail_body(i):
                off = i * 16
                x = ins[1][0, pl.ds(off, 16)]
                tok = ins[0][5, pl.ds(off, 16)]
                ps[0][0, pl.ds(off, 16)] = jnp.where(x >= 2, 4, 0)
                rs[0][0, pl.ds(off, 16)] = jnp.where(tok >= 2, tok, zero)
                ps[0][1, pl.ds(off, 16)] = zero
                rs[0][1, pl.ds(off, 16)] = zero
            pltpu.sync_copy(ps[0].at[pl.ds(0, 2)],
                            p_hbm.at[pl.ds(2048, 2), pl.ds(col0, HALF)])
            pltpu.sync_copy(rs[0].at[pl.ds(0, 2)],
                            r_hbm.at[pl.ds(2048, 2), pl.ds(col0, HALF)])

    return run


_RUN = _make_kernel()


def kernel(input_id_sequence):
    pt, rt = _RUN(input_id_sequence.T)
    return (pt.T, rt.T)
